# baseline (speedup 1.0000x reference)
import jax
import jax.numpy as jnp
from jax import lax
from jax.experimental import pallas as pl
from jax.experimental.pallas import tpu as pltpu
from jax.experimental.pallas import tpu_sc as plsc

N_NODES = 10000
N_PAD = 10112
N_EDGES = 320000
D = 128
D_EDGE = 16

NC, NS, L = 2, 16, 16
NW = NC * NS
CHUNK = 64
N_CHUNKS = N_EDGES // CHUNK
ROWS_PER_TILE = N_PAD // NS


EBLK = 8000


def _pre_node_body(nf_ref, w1_ref, w2_ref, be_ref, a_ref, b_ref):
    x = nf_ref[...]
    a_ref[...] = jnp.dot(x, w1_ref[...], preferred_element_type=jnp.float32) + be_ref[...]
    b_ref[...] = jnp.dot(x, w2_ref[...], preferred_element_type=jnp.float32)


_pre_node = pl.pallas_call(
    _pre_node_body,
    out_shape=(jax.ShapeDtypeStruct((N_NODES, D), jnp.float32),
               jax.ShapeDtypeStruct((N_NODES, D), jnp.float32)),
)


def _pre_edge_body(ef_ref, w3_ref, e_ref):
    e_ref[...] = jnp.dot(ef_ref[...].astype(jnp.bfloat16),
                         w3_ref[...].astype(jnp.bfloat16),
                         preferred_element_type=jnp.float32)


_pre_edge = pl.pallas_call(
    _pre_edge_body,
    grid=(N_EDGES // EBLK,),
    in_specs=[pl.BlockSpec((EBLK, D_EDGE), lambda i: (i, 0)),
              pl.BlockSpec((D_EDGE, D), lambda i: (0, 0))],
    out_specs=pl.BlockSpec((EBLK, D), lambda i: (i, 0)),
    out_shape=jax.ShapeDtypeStruct((N_EDGES, D), jnp.float32),
)


def _post_body(nf_ref, p_ref, w1_ref, w2_ref, bn_ref, o_ref):
    x = nf_ref[...]
    agg = p_ref[:N_NODES, :] + p_ref[N_PAD:N_PAD + N_NODES, :]
    o_ref[...] = (x + bn_ref[...]
                  + jnp.dot(x, w1_ref[...], preferred_element_type=jnp.float32)
                  + jnp.dot(agg, w2_ref[...], preferred_element_type=jnp.float32))


_post = pl.pallas_call(
    _post_body,
    out_shape=jax.ShapeDtypeStruct((N_NODES, D), jnp.float32),
)


def _sc_body(a_hbm, b_hbm, e_hbm, ei_hbm, out_hbm,
             si0, si1, ti0, ti1,
             ra0, ra1, rb0, rb1, re0, re1,
             acc,
             sa0, sa1, sb0, sb1, se0, se1, sp0, sp1):
    sidx = [si0, si1]
    tidx = [ti0, ti1]
    rows_a = [ra0, ra1]
    rows_b = [rb0, rb1]
    rows_e = [re0, re1]
    sem_a = [sa0, sa1]
    sem_b = [sb0, sb1]
    sem_e = [se0, se1]
    sem_i = [sp0, sp1]

    cid = lax.axis_index("c")
    sid = lax.axis_index("s")
    wid = sid * NC + cid

    z16 = jnp.zeros((L,), jnp.float32)

    def _zrow(i, _):
        for j in range(D // L):
            re0[i, pl.ds(j * L, L)] = z16
        return 0

    lax.fori_loop(0, CHUNK, _zrow, 0)
    base_row = sid * ROWS_PER_TILE
    nz = ROWS_PER_TILE // CHUNK
    for k in range(nz):
        pltpu.sync_copy(re0, acc.at[pl.ds(base_row + k * CHUNK, CHUNK), :])
    rem = ROWS_PER_TILE % CHUNK
    if rem:
        pltpu.sync_copy(re0.at[pl.ds(0, rem), :],
                        acc.at[pl.ds(base_row + nz * CHUNK, rem), :])
    plsc.subcore_barrier()

    n_mine = (N_CHUNKS // NW) + (wid < (N_CHUNKS % NW)).astype(jnp.int32)

    def chunk_base(k):
        return (wid + k * NW) * CHUNK

    def issue_idx(k, b):
        base = chunk_base(k)
        pltpu.async_copy(ei_hbm.at[0, pl.ds(base, CHUNK)], sidx[b], sem_i[b])
        pltpu.async_copy(ei_hbm.at[1, pl.ds(base, CHUNK)], tidx[b], sem_i[b])

    def wait_idx(b):
        pltpu.make_async_copy(ei_hbm.at[0, pl.ds(0, CHUNK)], sidx[b], sem_i[b]).wait()
        pltpu.make_async_copy(ei_hbm.at[1, pl.ds(0, CHUNK)], tidx[b], sem_i[b]).wait()

    def issue_gather(k, b):
        base = chunk_base(k)
        pltpu.async_copy(a_hbm.at[sidx[b]], rows_a[b], sem_a[b])
        pltpu.async_copy(b_hbm.at[tidx[b]], rows_b[b], sem_b[b])
        pltpu.async_copy(e_hbm.at[pl.ds(base, CHUNK), :], rows_e[b], sem_e[b])

    def wait_gather(b):
        pltpu.make_async_copy(a_hbm.at[pl.ds(0, CHUNK), :], rows_a[b], sem_a[b]).wait()
        pltpu.make_async_copy(b_hbm.at[pl.ds(0, CHUNK), :], rows_b[b], sem_b[b]).wait()
        pltpu.make_async_copy(e_hbm.at[pl.ds(0, CHUNK), :], rows_e[b], sem_e[b]).wait()

    issue_idx(0, 0)

    @pl.when(n_mine > 1)
    def _():
        issue_idx(1, 1)

    wait_idx(0)
    issue_gather(0, 0)

    max_outer = (N_CHUNKS // NW + 2) // 2

    def _outer(ko, _):
        for b in range(2):
            k = ko * 2 + b

            @pl.when(k < n_mine)
            def _it(k=k, b=b):
                @pl.when(k + 1 < n_mine)
                def _(k=k, b=b):
                    wait_idx(1 - b)
                    issue_gather(k + 1, 1 - b)

                wait_gather(b)

                def _crow(i, _c, b=b):
                    for j in range(D // L):
                        sl = pl.ds(j * L, L)
                        v = rows_a[b][i, sl] + rows_b[b][i, sl] + rows_e[b][i, sl]
                        rows_e[b][i, sl] = jnp.maximum(v, 0.0)
                    return 0

                lax.fori_loop(0, CHUNK, _crow, 0)
                pltpu.sync_copy(rows_e[b], acc.at[sidx[b]], add=True)

                @pl.when(k + 2 < n_mine)
                def _(k=k, b=b):
                    issue_idx(k + 2, b)
        return 0

    lax.fori_loop(0, max_outer, _outer, 0)

    plsc.subcore_barrier()
    pltpu.sync_copy(acc.at[pl.ds(base_row, ROWS_PER_TILE), :],
                    out_hbm.at[pl.ds(cid * N_PAD + base_row, ROWS_PER_TILE), :])


def _sc_agg(A, B, E, ei):
    idx_t = pltpu.VMEM((CHUNK,), jnp.int32)
    row_t = pltpu.VMEM((CHUNK, D), jnp.float32)
    sem_t = pltpu.SemaphoreType.DMA
    sc_call = pl.kernel(
        _sc_body,
        out_type=jax.ShapeDtypeStruct((NC * N_PAD, D), jnp.float32),
        mesh=plsc.VectorSubcoreMesh(core_axis_name="c", subcore_axis_name="s"),
        scratch_types=(
            [idx_t] * 4 + [row_t] * 6
            + [pltpu.VMEM_SHARED((N_PAD, D), jnp.float32)]
            + [sem_t] * 8
        ),
    )
    return sc_call(A, B, E, ei)


def kernel(node_features, edge_indices, edge_features, W_edge, b_edge, W_node, b_node):
    ei = edge_indices.astype(jnp.int32)
    A, B = _pre_node(node_features, W_edge[:D], W_edge[D:2 * D], b_edge.reshape(1, D))
    E = _pre_edge(edge_features, W_edge[2 * D:])
    partial = _sc_agg(A, B, E, ei)
    out = _post(node_features, partial, W_node[:D], W_node[D:], b_node.reshape(1, D))
    return (out, edge_indices, edge_features)


# speedup vs baseline: 1.0206x; 1.0206x over previous
import jax
import jax.numpy as jnp
from jax import lax
from jax.experimental import pallas as pl
from jax.experimental.pallas import tpu as pltpu
from jax.experimental.pallas import tpu_sc as plsc

N_NODES = 10000
N_PAD = 10112
N_EDGES = 320000
D = 128
D_EDGE = 16

NC, NS, L = 2, 16, 16
NW = NC * NS
CHUNK = 64
NE_H = N_EDGES // 2
N_CHUNKS = NE_H // CHUNK
ROWS_PER_TILE = N_PAD // NS


EBLK = 8000


def _pre_node_body(nf_ref, w1_ref, w2_ref, be_ref, a_ref, b_ref):
    x = nf_ref[...]
    a_ref[...] = jnp.dot(x, w1_ref[...], preferred_element_type=jnp.float32) + be_ref[...]
    b_ref[...] = jnp.dot(x, w2_ref[...], preferred_element_type=jnp.float32)


_pre_node = pl.pallas_call(
    _pre_node_body,
    out_shape=(jax.ShapeDtypeStruct((N_NODES, D), jnp.float32),
               jax.ShapeDtypeStruct((N_NODES, D), jnp.float32)),
)


def _pre_edge_body(ef_ref, w3_ref, e_ref):
    e_ref[...] = jnp.dot(ef_ref[...].astype(jnp.bfloat16),
                         w3_ref[...].astype(jnp.bfloat16),
                         preferred_element_type=jnp.float32)


_pre_edge = pl.pallas_call(
    _pre_edge_body,
    grid=(NE_H // EBLK,),
    in_specs=[pl.BlockSpec((EBLK, D_EDGE), lambda i: (i, 0)),
              pl.BlockSpec((D_EDGE, D), lambda i: (0, 0))],
    out_specs=pl.BlockSpec((EBLK, D), lambda i: (i, 0)),
    out_shape=jax.ShapeDtypeStruct((NE_H, D), jnp.float32),
)


def _post_body(nf_ref, p_ref, q_ref, w1_ref, w2_ref, bn_ref, o_ref):
    x = nf_ref[...]
    agg = (p_ref[:N_NODES, :] + p_ref[N_PAD:N_PAD + N_NODES, :]
           + q_ref[:N_NODES, :] + q_ref[N_PAD:N_PAD + N_NODES, :])
    o_ref[...] = (x + bn_ref[...]
                  + jnp.dot(x, w1_ref[...], preferred_element_type=jnp.float32)
                  + jnp.dot(agg, w2_ref[...], preferred_element_type=jnp.float32))


_post = pl.pallas_call(
    _post_body,
    out_shape=jax.ShapeDtypeStruct((N_NODES, D), jnp.float32),
)


def _sc_body(a_hbm, b_hbm, e_hbm, ei_hbm, out_hbm,
             si0, si1, ti0, ti1,
             ra0, ra1, rb0, rb1, re0, re1,
             acc,
             sa0, sa1, sb0, sb1, se0, se1, sp0, sp1):
    sidx = [si0, si1]
    tidx = [ti0, ti1]
    rows_a = [ra0, ra1]
    rows_b = [rb0, rb1]
    rows_e = [re0, re1]
    sem_a = [sa0, sa1]
    sem_b = [sb0, sb1]
    sem_e = [se0, se1]
    sem_i = [sp0, sp1]

    cid = lax.axis_index("c")
    sid = lax.axis_index("s")
    wid = sid * NC + cid

    z16 = jnp.zeros((L,), jnp.float32)

    def _zrow(i, _):
        for j in range(D // L):
            re0[i, pl.ds(j * L, L)] = z16
        return 0

    lax.fori_loop(0, CHUNK, _zrow, 0)
    base_row = sid * ROWS_PER_TILE
    nz = ROWS_PER_TILE // CHUNK
    for k in range(nz):
        pltpu.sync_copy(re0, acc.at[pl.ds(base_row + k * CHUNK, CHUNK), :])
    rem = ROWS_PER_TILE % CHUNK
    if rem:
        pltpu.sync_copy(re0.at[pl.ds(0, rem), :],
                        acc.at[pl.ds(base_row + nz * CHUNK, rem), :])
    plsc.subcore_barrier()

    n_mine = (N_CHUNKS // NW) + (wid < (N_CHUNKS % NW)).astype(jnp.int32)

    def chunk_base(k):
        return (wid + k * NW) * CHUNK

    def issue_idx(k, b):
        base = chunk_base(k)
        pltpu.async_copy(ei_hbm.at[0, pl.ds(base, CHUNK)], sidx[b], sem_i[b])
        pltpu.async_copy(ei_hbm.at[1, pl.ds(base, CHUNK)], tidx[b], sem_i[b])

    def wait_idx(b):
        pltpu.make_async_copy(ei_hbm.at[0, pl.ds(0, CHUNK)], sidx[b], sem_i[b]).wait()
        pltpu.make_async_copy(ei_hbm.at[1, pl.ds(0, CHUNK)], tidx[b], sem_i[b]).wait()

    def issue_gather(k, b):
        base = chunk_base(k)
        pltpu.async_copy(a_hbm.at[sidx[b]], rows_a[b], sem_a[b])
        pltpu.async_copy(b_hbm.at[tidx[b]], rows_b[b], sem_b[b])
        pltpu.async_copy(e_hbm.at[pl.ds(base, CHUNK), :], rows_e[b], sem_e[b])

    def wait_gather(b):
        pltpu.make_async_copy(a_hbm.at[pl.ds(0, CHUNK), :], rows_a[b], sem_a[b]).wait()
        pltpu.make_async_copy(b_hbm.at[pl.ds(0, CHUNK), :], rows_b[b], sem_b[b]).wait()
        pltpu.make_async_copy(e_hbm.at[pl.ds(0, CHUNK), :], rows_e[b], sem_e[b]).wait()

    issue_idx(0, 0)

    @pl.when(n_mine > 1)
    def _():
        issue_idx(1, 1)

    wait_idx(0)
    issue_gather(0, 0)

    max_outer = (N_CHUNKS // NW + 2) // 2

    def _outer(ko, _):
        for b in range(2):
            k = ko * 2 + b

            @pl.when(k < n_mine)
            def _it(k=k, b=b):
                @pl.when(k + 1 < n_mine)
                def _(k=k, b=b):
                    wait_idx(1 - b)
                    issue_gather(k + 1, 1 - b)

                wait_gather(b)

                def _crow(i, _c, b=b):
                    for j in range(D // L):
                        sl = pl.ds(j * L, L)
                        v = rows_a[b][i, sl] + rows_b[b][i, sl] + rows_e[b][i, sl]
                        rows_e[b][i, sl] = jnp.maximum(v, 0.0)
                    return 0

                lax.fori_loop(0, CHUNK, _crow, 0)
                pltpu.sync_copy(rows_e[b], acc.at[sidx[b]], add=True)

                @pl.when(k + 2 < n_mine)
                def _(k=k, b=b):
                    issue_idx(k + 2, b)
        return 0

    lax.fori_loop(0, max_outer, _outer, 0)

    plsc.subcore_barrier()
    pltpu.sync_copy(acc.at[pl.ds(base_row, ROWS_PER_TILE), :],
                    out_hbm.at[pl.ds(cid * N_PAD + base_row, ROWS_PER_TILE), :])


def _sc_agg(A, B, E, ei):
    idx_t = pltpu.VMEM((CHUNK,), jnp.int32)
    row_t = pltpu.VMEM((CHUNK, D), jnp.float32)
    sem_t = pltpu.SemaphoreType.DMA
    sc_call = pl.kernel(
        _sc_body,
        out_type=jax.ShapeDtypeStruct((NC * N_PAD, D), jnp.float32),
        mesh=plsc.VectorSubcoreMesh(core_axis_name="c", subcore_axis_name="s"),
        scratch_types=(
            [idx_t] * 4 + [row_t] * 6
            + [pltpu.VMEM_SHARED((N_PAD, D), jnp.float32)]
            + [sem_t] * 8
        ),
    )
    return sc_call(A, B, E, ei)


def kernel(node_features, edge_indices, edge_features, W_edge, b_edge, W_node, b_node):
    ei = edge_indices.astype(jnp.int32)
    A, B = _pre_node(node_features, W_edge[:D], W_edge[D:2 * D], b_edge.reshape(1, D))
    E1 = _pre_edge(edge_features[:NE_H], W_edge[2 * D:])
    p1 = _sc_agg(A, B, E1, ei[:, :NE_H])
    E2 = _pre_edge(edge_features[NE_H:], W_edge[2 * D:])
    p2 = _sc_agg(A, B, E2, ei[:, NE_H:])
    out = _post(node_features, p1, p2, W_node[:D], W_node[D:], b_node.reshape(1, D))
    return (out, edge_indices, edge_features)
